# baseline (speedup 1.0000x reference)
import functools

import jax
import jax.numpy as jnp
from jax import lax
from jax.experimental import pallas as pl
from jax.experimental.pallas import tpu as pltpu
from jax.experimental.pallas import tpu_sc as plsc

NC = 2
NS = 16
L = 16

B = 4096
N = 1000
D = 16
LLEN = 50

NW = NC * NS
BPW = B // NW
GPB = BPW // L

NBUF = 10
NOUT = LLEN // NBUF

N_STRIDE = 65536
DT_STRIDE = 32768
DM_STRIDE = 128
BT_STRIDE = 1024

_mesh = plsc.VectorSubcoreMesh(
    core_axis_name="c", subcore_axis_name="s", num_cores=NC, num_subcores=NS
)


@functools.partial(
    pl.kernel,
    out_type=jax.ShapeDtypeStruct((D, B), jnp.float32),
    mesh=_mesh,
    compiler_params=pltpu.CompilerParams(
        needs_layout_passes=False, use_tc_tiling_on_sc=False
    ),
    scratch_types=[
        pltpu.VMEM((BPW, LLEN), jnp.int32),
        pltpu.VMEM((LLEN, BPW), jnp.int32),
        pltpu.VMEM((NBUF, D * BPW), jnp.int32),
        pltpu.VMEM((NBUF, D * BPW), jnp.float32),
        pltpu.VMEM((D, BPW), jnp.float32),
        [pltpu.SemaphoreType.DMA] * NBUF,
    ],
)
def _sc_embed_sum(flat_hbm, inpt_hbm, out_hbm, idx_raw, idx_t, idxr, datr, acc, sems):
    wid = lax.axis_index("s") * NC + lax.axis_index("c")
    base = wid * BPW

    pltpu.sync_copy(inpt_hbm.at[pl.ds(base, BPW)], idx_raw)

    lanes = lax.iota(jnp.int32, L)

    def build_l(l, _):
        cols = jnp.full((L,), 0, jnp.int32) + l

        def build_g(g, _):
            rows = lanes + g * L
            idx_t[l, pl.ds(g * L, L)] = plsc.load_gather(idx_raw, [rows, cols])
            return 0

        return lax.fori_loop(0, GPB, build_g, 0, unroll=True)

    lax.fori_loop(0, LLEN, build_l, 0)

    zeros = jnp.zeros((L,), jnp.float32)

    def zero_i(i, _):
        acc[i // GPB, pl.ds((i % GPB) * L, L)] = zeros
        return 0

    lax.fori_loop(0, D * GPB, zero_i, 0)

    def build_addr(j, l):
        for g in range(GPB):
            n16 = idx_t[l, pl.ds(g * L, L)]
            vbase = n16 * N_STRIDE + wid * BT_STRIDE + (lanes + g * L)
            for d in range(D):
                doff = (d // 8) * DT_STRIDE + (d % 8) * DM_STRIDE
                idxr[j, pl.ds(d * BPW + g * L, L)] = vbase + doff

    def fire(j):
        for d in range(D):
            pltpu.async_copy(
                flat_hbm.at[idxr.at[j, pl.ds(d * BPW, BPW)]],
                datr.at[j, pl.ds(d * BPW, BPW)],
                sems[j],
            )

    def wait(j):
        pltpu.make_async_copy(
            out_hbm.at[0, pl.ds(0, D * BPW)], datr.at[j], sems[j]
        ).wait()

    def accumulate(j):
        def acc_i(i, _):
            d, g = i // GPB, i % GPB
            plsc.addupdate(
                acc.at[d, pl.ds(g * L, L)], datr[j, pl.ds(d * BPW + g * L, L)]
            )
            return 0

        lax.fori_loop(0, D * GPB, acc_i, 0, unroll=4)

    def outer(p, _):
        for j in range(NBUF):
            l_fire = p * NBUF + j

            @pl.when(l_fire >= NBUF)
            def _():
                wait(j)
                accumulate(j)

            @pl.when(l_fire < LLEN)
            def _():
                build_addr(j, l_fire)
                fire(j)

        return 0

    lax.fori_loop(0, NOUT + 1, outer, 0)

    pltpu.sync_copy(acc, out_hbm.at[:, pl.ds(base, BPW)])


def kernel(scdata, inpt):
    v = scdata.reshape(NW, BPW, N, D // 8, 8)
    v = v.transpose(2, 3, 0, 4, 1)
    flat = v.reshape(B * N * D)
    out_t = _sc_embed_sum(flat, inpt.astype(jnp.int32))
    return out_t.T


# speedup vs baseline: 1.0171x; 1.0171x over previous
import functools

import jax
import jax.numpy as jnp
from jax import lax
from jax.experimental import pallas as pl
from jax.experimental.pallas import tpu as pltpu
from jax.experimental.pallas import tpu_sc as plsc

NC = 2
NS = 16
L = 16

B = 4096
N = 1000
D = 16
LLEN = 50

NW = NC * NS
BPW = B // NW
GPB = BPW // L

NBUF = 5
NOUT = LLEN // NBUF

N_STRIDE = 65536
DT_STRIDE = 32768
DM_STRIDE = 128
BT_STRIDE = 1024

_mesh = plsc.VectorSubcoreMesh(
    core_axis_name="c", subcore_axis_name="s", num_cores=NC, num_subcores=NS
)


@functools.partial(
    pl.kernel,
    out_type=jax.ShapeDtypeStruct((D, B), jnp.float32),
    mesh=_mesh,
    compiler_params=pltpu.CompilerParams(
        needs_layout_passes=False, use_tc_tiling_on_sc=False
    ),
    scratch_types=[
        pltpu.VMEM((BPW, LLEN), jnp.int32),
        pltpu.VMEM((LLEN, BPW), jnp.int32),
        pltpu.VMEM((NBUF, D, BPW), jnp.int32),
        pltpu.VMEM((NBUF, D, BPW), jnp.float32),
        pltpu.VMEM((D, BPW), jnp.float32),
        [pltpu.SemaphoreType.DMA] * NBUF,
    ],
)
def _sc_embed_sum(flat_hbm, inpt_hbm, out_hbm, idx_raw, idx_t, idxr, datr, acc, sems):
    wid = lax.axis_index("s") * NC + lax.axis_index("c")
    base = wid * BPW

    pltpu.sync_copy(inpt_hbm.at[pl.ds(base, BPW)], idx_raw)

    lanes = lax.iota(jnp.int32, L)

    def build_l(l, _):
        cols = jnp.full((L,), 0, jnp.int32) + l

        def build_g(g, _):
            rows = lanes + g * L
            idx_t[l, pl.ds(g * L, L)] = plsc.load_gather(idx_raw, [rows, cols])
            return 0

        return lax.fori_loop(0, GPB, build_g, 0, unroll=True)

    lax.fori_loop(0, LLEN, build_l, 0)

    zeros = jnp.zeros((L,), jnp.float32)

    def zero_i(i, _):
        acc[i // GPB, pl.ds((i % GPB) * L, L)] = zeros
        return 0

    lax.fori_loop(0, D * GPB, zero_i, 0)

    def build_addr(j, l):
        for g in range(GPB):
            n16 = idx_t[l, pl.ds(g * L, L)]
            vbase = n16 * N_STRIDE + wid * BT_STRIDE + (lanes + g * L)
            for d in range(D):
                doff = (d // 8) * DT_STRIDE + (d % 8) * DM_STRIDE
                idxr[j, d, pl.ds(g * L, L)] = vbase + doff

    def fire(j):
        for d in range(D):
            pltpu.async_copy(flat_hbm.at[idxr.at[j, d]], datr.at[j, d], sems[j])

    def wait(j):
        pltpu.make_async_copy(
            out_hbm.at[:, pl.ds(0, BPW)], datr.at[j], sems[j]
        ).wait()

    def accumulate(j):
        def acc_i(i, _):
            d, g = i // GPB, i % GPB
            plsc.addupdate(
                acc.at[d, pl.ds(g * L, L)], datr[j, d, pl.ds(g * L, L)]
            )
            return 0

        lax.fori_loop(0, D * GPB, acc_i, 0, unroll=4)

    def outer(p, _):
        for j in range(NBUF):
            l_fire = p * NBUF + j

            @pl.when(l_fire >= NBUF)
            def _():
                wait(j)
                accumulate(j)

            @pl.when(l_fire < LLEN)
            def _():
                build_addr(j, l_fire)
                fire(j)

        return 0

    lax.fori_loop(0, NOUT + 1, outer, 0)

    pltpu.sync_copy(acc, out_hbm.at[:, pl.ds(base, BPW)])


def kernel(scdata, inpt):
    v = scdata.reshape(NW, BPW, N, D // 8, 8)
    v = v.transpose(2, 3, 0, 4, 1)
    flat = v.reshape(B * N * D)
    out_t = _sc_embed_sum(flat, inpt.astype(jnp.int32))
    return out_t.T
